# Initial kernel scaffold; baseline (speedup 1.0000x reference)
import jax
import jax.numpy as jnp
from jax.experimental import pallas as pl


def kernel(x, edge_index, edge_attr, weight_node, weight_edge, weight_triplet_att, weight_scale, bias):
    raise NotImplementedError("write your pallas kernel here")


# speedup vs baseline: 2.8720x; 2.8720x over previous
import jax
import jax.numpy as jnp
from jax.experimental import pallas as pl
from jax.experimental.pallas import tpu as pltpu

N = 10000
E = 320000
D = 128
H = 3
HD = H * D
NP = 10240
TN = 2048
NCH = NP // TN
TE = 256
F32 = jnp.float32


def _k0_body(x_ref, wn_ref, wi_ref, xp_ref, ai_ref):
    xp = jnp.dot(x_ref[...], wn_ref[...], preferred_element_type=F32)
    xp_ref[...] = xp
    wi = wi_ref[...]
    cols = [
        jnp.sum(xp[:, h * D:(h + 1) * D] * wi[h][None, :], axis=1, keepdims=True)
        for h in range(H)
    ]
    cols.append(jnp.zeros((xp.shape[0], 8 - H), F32))
    ai_ref[...] = jnp.concatenate(cols, axis=1)


def _k1_body(src_ref, dst_ref, dstr_ref, ea_ref, we_ref, watt_ref,
             xp_ref, ai_ref, s_ref, den_ref):
    e = pl.program_id(0)

    @pl.when(e == 0)
    def _():
        s_ref[...] = jnp.zeros_like(s_ref)
        den_ref[...] = jnp.zeros_like(den_ref)

    ep = jnp.dot(ea_ref[...], we_ref[...], preferred_element_type=F32)
    src = src_ref[...]
    dst = dst_ref[...]

    xj = jnp.zeros((TE, HD), F32)
    aid = jnp.zeros((TE, 8), F32)
    for c in range(NCH):
        col = jax.lax.broadcasted_iota(jnp.int32, (TE, TN), 1) + c * TN
        os_ = (col == src).astype(F32)
        od = (col == dst).astype(F32)
        xp_c = xp_ref[c * TN:(c + 1) * TN, :]
        ai_c = ai_ref[c * TN:(c + 1) * TN, :]
        xj = xj + jnp.dot(os_, xp_c, preferred_element_type=F32)
        aid = aid + jnp.dot(od, ai_c, preferred_element_type=F32)

    watt = watt_ref[...]
    we_att = watt[0:H, D:2 * D]
    wj_att = watt[0:H, 2 * D:3 * D]
    alist = []
    for h in range(H):
        eph = ep[:, h * D:(h + 1) * D]
        xjh = xj[:, h * D:(h + 1) * D]
        a_e = jnp.sum(eph * we_att[h][None, :], axis=1, keepdims=True)
        a_j = jnp.sum(xjh * wj_att[h][None, :], axis=1, keepdims=True)
        alist.append(aid[:, h:h + 1] + a_e + a_j)
    alpha = jnp.concatenate(alist, axis=1)
    alpha = jnp.where(alpha >= 0, alpha, 0.2 * alpha)
    ex = jnp.exp(alpha)

    glist = [
        ex[:, h:h + 1] * ep[:, h * D:(h + 1) * D] * xj[:, h * D:(h + 1) * D]
        for h in range(H)
    ]
    g = jnp.concatenate(glist, axis=1)
    ex8 = jnp.concatenate([ex, jnp.zeros((TE, 8 - H), F32)], axis=1)

    dstr = dstr_ref[...]
    for c in range(NCH):
        colt = jax.lax.broadcasted_iota(jnp.int32, (TN, TE), 0) + c * TN
        odt = (colt == dstr).astype(F32)
        s_ref[c * TN:(c + 1) * TN, :] = s_ref[c * TN:(c + 1) * TN, :] + jnp.dot(
            odt, g, preferred_element_type=F32)
        den_ref[c * TN:(c + 1) * TN, :] = den_ref[c * TN:(c + 1) * TN, :] + jnp.dot(
            odt, ex8, preferred_element_type=F32)


def _k2_body(s_ref, den_ref, ws_ref, b_ref, o_ref):
    den = den_ref[...][:, 0:H]
    r = 1.0 / jnp.maximum(den, 1e-16)
    s = s_ref[...]
    parts = [s[:, h * D:(h + 1) * D] * r[:, h:h + 1] for h in range(H)]
    a = jnp.concatenate(parts, axis=1)
    o_ref[...] = jnp.dot(a, ws_ref[...], preferred_element_type=F32) + b_ref[...]


def kernel(x, edge_index, edge_attr, weight_node, weight_edge,
           weight_triplet_att, weight_scale, bias):
    x = x.astype(F32)
    edge_attr = edge_attr.astype(F32)
    weight_node = weight_node.astype(F32)
    weight_edge = weight_edge.astype(F32)
    weight_scale = weight_scale.astype(F32)
    bias = bias.astype(F32)

    src = edge_index[0].astype(jnp.int32)
    dst = edge_index[1].astype(jnp.int32)
    src_col = src.reshape(E, 1)
    dst_col = dst.reshape(E, 1)
    dst_row = dst.reshape(1, E)

    x_pad = jnp.pad(x, ((0, NP - N), (0, 0)))
    wi_pad = jnp.pad(weight_triplet_att[0, :, 0:D].astype(F32), ((0, 8 - H), (0, 0)))
    watt_pad = jnp.pad(weight_triplet_att[0].astype(F32), ((0, 8 - H), (0, 0)))

    xp, ai8 = pl.pallas_call(
        _k0_body,
        grid=(NP // TN,),
        in_specs=[
            pl.BlockSpec((TN, D), lambda i: (i, i - i)),
            pl.BlockSpec((D, HD), lambda i: (i - i, i - i)),
            pl.BlockSpec((8, D), lambda i: (i - i, i - i)),
        ],
        out_specs=[
            pl.BlockSpec((TN, HD), lambda i: (i, i - i)),
            pl.BlockSpec((TN, 8), lambda i: (i, i - i)),
        ],
        out_shape=[
            jax.ShapeDtypeStruct((NP, HD), F32),
            jax.ShapeDtypeStruct((NP, 8), F32),
        ],
    )(x_pad, weight_node, wi_pad)

    s_acc, den_acc = pl.pallas_call(
        _k1_body,
        grid=(E // TE,),
        in_specs=[
            pl.BlockSpec((TE, 1), lambda e: (e, e - e)),
            pl.BlockSpec((TE, 1), lambda e: (e, e - e)),
            pl.BlockSpec((1, TE), lambda e: (e - e, e)),
            pl.BlockSpec((TE, 16), lambda e: (e, e - e)),
            pl.BlockSpec((16, HD), lambda e: (e - e, e - e)),
            pl.BlockSpec((8, 3 * D), lambda e: (e - e, e - e)),
            pl.BlockSpec((NP, HD), lambda e: (e - e, e - e)),
            pl.BlockSpec((NP, 8), lambda e: (e - e, e - e)),
        ],
        out_specs=[
            pl.BlockSpec((NP, HD), lambda e: (e - e, e - e)),
            pl.BlockSpec((NP, 8), lambda e: (e - e, e - e)),
        ],
        out_shape=[
            jax.ShapeDtypeStruct((NP, HD), F32),
            jax.ShapeDtypeStruct((NP, 8), F32),
        ],
        compiler_params=pltpu.CompilerParams(
            dimension_semantics=("arbitrary",)),
    )(src_col, dst_col, dst_row, edge_attr, weight_edge, watt_pad, xp, ai8)

    out = pl.pallas_call(
        _k2_body,
        grid=(NP // TN,),
        in_specs=[
            pl.BlockSpec((TN, HD), lambda i: (i, i - i)),
            pl.BlockSpec((TN, 8), lambda i: (i, i - i)),
            pl.BlockSpec((HD, D), lambda i: (i - i, i - i)),
            pl.BlockSpec((1, D), lambda i: (i - i, i - i)),
        ],
        out_specs=pl.BlockSpec((TN, D), lambda i: (i, i - i)),
        out_shape=jax.ShapeDtypeStruct((NP, D), F32),
    )(s_acc, den_acc, weight_scale, bias.reshape(1, D))

    return out[:N]


# speedup vs baseline: 2.9056x; 1.0117x over previous
import jax
import jax.numpy as jnp
from jax.experimental import pallas as pl
from jax.experimental.pallas import tpu as pltpu

N = 10000
E = 320000
D = 128
H = 3
HD = H * D
NP = 10240
TN = 2048
NCH = NP // TN
TE = 256
F32 = jnp.float32


def _k0_body(x_ref, wn_ref, watt_ref, xp_ref, ai_ref):
    xp = jnp.dot(x_ref[...], wn_ref[...], preferred_element_type=F32)
    xp_ref[...] = xp
    watt = watt_ref[...]
    cols = [
        jnp.sum(xp[:, h * D:(h + 1) * D] * watt[h, 0:D][None, :],
                axis=1, keepdims=True)
        for h in range(H)
    ] + [
        jnp.sum(xp[:, h * D:(h + 1) * D] * watt[h, 2 * D:3 * D][None, :],
                axis=1, keepdims=True)
        for h in range(H)
    ]
    cols.append(jnp.zeros((xp.shape[0], 8 - 2 * H), F32))
    ai_ref[...] = jnp.concatenate(cols, axis=1)


def _k1_body(src_ref, dst_ref, dstr_ref, ea_ref, we_ref, watt_ref,
             xp_ref, ai_ref, s_ref, den_ref):
    e = pl.program_id(0)

    @pl.when(e == 0)
    def _():
        s_ref[...] = jnp.zeros_like(s_ref)
        den_ref[...] = jnp.zeros_like(den_ref)

    ep = jnp.dot(ea_ref[...], we_ref[...], preferred_element_type=F32)
    src = src_ref[...]
    dst = dst_ref[...]

    xj = jnp.zeros((TE, HD), F32)
    aid = jnp.zeros((TE, 8), F32)
    ajs = jnp.zeros((TE, 8), F32)
    for c in range(NCH):
        col = jax.lax.broadcasted_iota(jnp.int32, (TE, TN), 1) + c * TN
        ms = col == src
        md = col == dst
        xp_c = xp_ref[c * TN:(c + 1) * TN, :]
        ai_c = ai_ref[c * TN:(c + 1) * TN, :]
        xj = xj + jnp.dot(ms.astype(jnp.bfloat16), xp_c,
                          preferred_element_type=F32)
        aid = aid + jnp.dot(md.astype(F32), ai_c, preferred_element_type=F32)
        ajs = ajs + jnp.dot(ms.astype(F32), ai_c, preferred_element_type=F32)

    watt = watt_ref[...]
    we_att = watt[0:H, D:2 * D]
    alist = []
    for h in range(H):
        eph = ep[:, h * D:(h + 1) * D]
        a_e = jnp.sum(eph * we_att[h][None, :], axis=1, keepdims=True)
        alist.append(aid[:, h:h + 1] + ajs[:, H + h:H + h + 1] + a_e)
    alpha = jnp.concatenate(alist, axis=1)
    alpha = jnp.where(alpha >= 0, alpha, 0.2 * alpha)
    ex = jnp.exp(alpha)

    glist = [
        ex[:, h:h + 1] * ep[:, h * D:(h + 1) * D] * xj[:, h * D:(h + 1) * D]
        for h in range(H)
    ]
    g = jnp.concatenate(glist, axis=1).astype(jnp.bfloat16)
    ex8 = jnp.concatenate([ex, jnp.zeros((TE, 8 - H), F32)], axis=1)

    dstr = dstr_ref[...]
    for c in range(NCH):
        colt = jax.lax.broadcasted_iota(jnp.int32, (TN, TE), 0) + c * TN
        mt = colt == dstr
        s_ref[c * TN:(c + 1) * TN, :] = s_ref[c * TN:(c + 1) * TN, :] + jnp.dot(
            mt.astype(jnp.bfloat16), g, preferred_element_type=F32)
        den_ref[c * TN:(c + 1) * TN, :] = den_ref[c * TN:(c + 1) * TN, :] + jnp.dot(
            mt.astype(F32), ex8, preferred_element_type=F32)


def _k2_body(s_ref, den_ref, ws_ref, b_ref, o_ref):
    den = den_ref[...][:, 0:H]
    r = 1.0 / jnp.maximum(den, 1e-16)
    s = s_ref[...]
    parts = [s[:, h * D:(h + 1) * D] * r[:, h:h + 1] for h in range(H)]
    a = jnp.concatenate(parts, axis=1)
    o_ref[...] = jnp.dot(a, ws_ref[...], preferred_element_type=F32) + b_ref[...]


def kernel(x, edge_index, edge_attr, weight_node, weight_edge,
           weight_triplet_att, weight_scale, bias):
    x = x.astype(F32)
    edge_attr = edge_attr.astype(F32)
    weight_node = weight_node.astype(F32)
    weight_edge = weight_edge.astype(F32)
    weight_scale = weight_scale.astype(F32)
    bias = bias.astype(F32)

    src = edge_index[0].astype(jnp.int32)
    dst = edge_index[1].astype(jnp.int32)
    src_col = src.reshape(E, 1)
    dst_col = dst.reshape(E, 1)
    dst_row = dst.reshape(1, E)

    x_pad = jnp.pad(x, ((0, NP - N), (0, 0)))
    watt_pad = jnp.pad(weight_triplet_att[0].astype(F32), ((0, 8 - H), (0, 0)))

    xp, ai8 = pl.pallas_call(
        _k0_body,
        grid=(NP // TN,),
        in_specs=[
            pl.BlockSpec((TN, D), lambda i: (i, i - i)),
            pl.BlockSpec((D, HD), lambda i: (i - i, i - i)),
            pl.BlockSpec((8, 3 * D), lambda i: (i - i, i - i)),
        ],
        out_specs=[
            pl.BlockSpec((TN, HD), lambda i: (i, i - i)),
            pl.BlockSpec((TN, 8), lambda i: (i, i - i)),
        ],
        out_shape=[
            jax.ShapeDtypeStruct((NP, HD), F32),
            jax.ShapeDtypeStruct((NP, 8), F32),
        ],
    )(x_pad, weight_node, watt_pad)

    xp_bf = xp.astype(jnp.bfloat16)

    s_acc, den_acc = pl.pallas_call(
        _k1_body,
        grid=(E // TE,),
        in_specs=[
            pl.BlockSpec((TE, 1), lambda e: (e, e - e)),
            pl.BlockSpec((TE, 1), lambda e: (e, e - e)),
            pl.BlockSpec((1, TE), lambda e: (e - e, e)),
            pl.BlockSpec((TE, 16), lambda e: (e, e - e)),
            pl.BlockSpec((16, HD), lambda e: (e - e, e - e)),
            pl.BlockSpec((8, 3 * D), lambda e: (e - e, e - e)),
            pl.BlockSpec((NP, HD), lambda e: (e - e, e - e)),
            pl.BlockSpec((NP, 8), lambda e: (e - e, e - e)),
        ],
        out_specs=[
            pl.BlockSpec((NP, HD), lambda e: (e - e, e - e)),
            pl.BlockSpec((NP, 8), lambda e: (e - e, e - e)),
        ],
        out_shape=[
            jax.ShapeDtypeStruct((NP, HD), F32),
            jax.ShapeDtypeStruct((NP, 8), F32),
        ],
        compiler_params=pltpu.CompilerParams(
            dimension_semantics=("arbitrary",)),
    )(src_col, dst_col, dst_row, edge_attr, weight_edge, watt_pad, xp_bf, ai8)

    out = pl.pallas_call(
        _k2_body,
        grid=(NP // TN,),
        in_specs=[
            pl.BlockSpec((TN, HD), lambda i: (i, i - i)),
            pl.BlockSpec((TN, 8), lambda i: (i, i - i)),
            pl.BlockSpec((HD, D), lambda i: (i - i, i - i)),
            pl.BlockSpec((1, D), lambda i: (i - i, i - i)),
        ],
        out_specs=pl.BlockSpec((TN, D), lambda i: (i, i - i)),
        out_shape=jax.ShapeDtypeStruct((NP, D), F32),
    )(s_acc, den_acc, weight_scale, bias.reshape(1, D))

    return out[:N]
